# baseline (speedup 1.0000x reference)
import functools

import jax
import jax.numpy as jnp
from jax import lax
from jax.experimental import pallas as pl
from jax.experimental.pallas import tpu as pltpu
from jax.experimental.pallas import tpu_sc as plsc

_B, _H, _W = 16, 2048, 2048
_K = 64
_LANE = 128
_CPR = _W // _LANE
_CH = _B * _H * _CPR
_NW = 32
_EPT = _B * _K // _NW
_RB = 1024
_NRB = _H // _RB

_mesh = plsc.VectorSubcoreMesh(core_axis_name="c", subcore_axis_name="s",
                               num_cores=2, num_subcores=16)


@functools.partial(
    pl.kernel,
    out_type=jax.ShapeDtypeStruct((_B * _K, _LANE), jnp.float32),
    mesh=_mesh,
    compiler_params=pltpu.CompilerParams(needs_layout_passes=False),
    scratch_types=[
        pltpu.VMEM((_EPT,), jnp.int32),
        pltpu.VMEM((_EPT,), jnp.int32),
        pltpu.VMEM((_EPT,), jnp.float32),
        pltpu.VMEM((_EPT, _LANE), jnp.float32),
        pltpu.SemaphoreType.DMA,
    ],
)
def _sc_patch(Y2, idx_hbm, off_hbm, val_hbm, patch, idx_v, off_v, val_v,
              chunks_v, sem_g):
    wid = lax.axis_index("s") * 2 + lax.axis_index("c")
    pltpu.sync_copy(idx_hbm.at[wid], idx_v)
    pltpu.sync_copy(off_hbm.at[wid], off_v)
    pltpu.sync_copy(val_hbm.at[wid], val_v)
    pltpu.async_copy(Y2.at[idx_v], chunks_v, sem_g).wait()
    for g in range(_EPT // 16):
        rows = lax.iota(jnp.int32, 16) + g * 16
        cols = off_v[pl.ds(g * 16, 16)]
        vals = val_v[pl.ds(g * 16, 16)]
        plsc.addupdate_scatter(chunks_v, [rows, cols], vals)
    pltpu.sync_copy(chunks_v, patch.at[pl.ds(wid * _EPT, _EPT)])


def _tc_body(y_s, cc_s, lo_s, hi_s, P2_ref, Yb_ref, out_ref):
    b = pl.program_id(0)
    rb = pl.program_id(1)
    out_ref[...] = Yb_ref[...]
    lanecc = jax.lax.broadcasted_iota(jnp.int32, (1, _W), 1) // _LANE

    def upd(k, carry):
        local = y_s[k] - rb * _RB
        tiled = P2_ref[pl.ds(b * _K + k, 1), :]
        row = Yb_ref[0, pl.ds(local, 1), :]
        out_ref[0, pl.ds(local, 1), :] = jnp.where(lanecc == cc_s[k],
                                                   tiled, row)
        return carry

    jax.lax.fori_loop(lo_s[rb], hi_s[rb], upd, 0)


def kernel(Y, X, y_idx, x_idx):
    bb = jnp.repeat(jnp.arange(_B, dtype=jnp.int32), _K)
    yk = jnp.tile(y_idx, (_B,))
    xk = jnp.tile(x_idx, (_B,))
    chunk_e = ((bb * _H + yk) * _CPR + xk // _LANE).reshape(_NW, _EPT)
    off_e = (xk % _LANE).reshape(_NW, _EPT)
    val_e = X.reshape(_NW, _EPT)

    patch = _sc_patch(Y.reshape(_CH, _LANE), chunk_e, off_e, val_e)
    patch_tiled = jnp.tile(patch, (1, _CPR))

    edges = jnp.arange(_NRB, dtype=jnp.int32) * _RB
    lo = jnp.searchsorted(y_idx, edges).astype(jnp.int32)
    hi = jnp.searchsorted(y_idx, edges + _RB).astype(jnp.int32)
    cc = (x_idx // _LANE).astype(jnp.int32)

    out = pl.pallas_call(
        _tc_body,
        grid_spec=pltpu.PrefetchScalarGridSpec(
            num_scalar_prefetch=4,
            grid=(_B, _NRB),
            in_specs=[
                pl.BlockSpec((_B * _K, _W), lambda b, rb, *_: (0, 0)),
                pl.BlockSpec((1, _RB, _W), lambda b, rb, *_: (b, rb, 0)),
            ],
            out_specs=pl.BlockSpec((1, _RB, _W), lambda b, rb, *_: (b, rb, 0)),
        ),
        out_shape=jax.ShapeDtypeStruct((_B, _H, _W), jnp.float32),
    )(y_idx, cc, lo, hi, patch_tiled, Y)
    return out


# speedup vs baseline: 2.4590x; 2.4590x over previous
import functools

import jax
import jax.numpy as jnp
from jax import lax
from jax.experimental import pallas as pl
from jax.experimental.pallas import tpu as pltpu
from jax.experimental.pallas import tpu_sc as plsc

_B, _H, _W = 16, 2048, 2048
_K = 64
_NW = 32
_EPT = _B * _K // _NW
_RB = 1024
_NRB = _H // _RB

_mesh = plsc.VectorSubcoreMesh(core_axis_name="c", subcore_axis_name="s",
                               num_cores=2, num_subcores=16)


@functools.partial(
    pl.kernel,
    out_type=jax.ShapeDtypeStruct((_B * _K, _W), jnp.float32),
    mesh=_mesh,
    compiler_params=pltpu.CompilerParams(needs_layout_passes=False),
    scratch_types=[
        pltpu.VMEM((_EPT,), jnp.int32),
        pltpu.VMEM((_EPT,), jnp.int32),
        pltpu.VMEM((_EPT,), jnp.float32),
        pltpu.VMEM((_EPT, _W), jnp.float32),
        pltpu.SemaphoreType.DMA,
    ],
)
def _sc_patch(Y2, idx_hbm, off_hbm, val_hbm, patch, idx_v, off_v, val_v,
              rows_v, sem_g):
    wid = lax.axis_index("s") * 2 + lax.axis_index("c")
    pltpu.sync_copy(idx_hbm.at[wid], idx_v)
    pltpu.sync_copy(off_hbm.at[wid], off_v)
    pltpu.sync_copy(val_hbm.at[wid], val_v)
    pltpu.async_copy(Y2.at[idx_v], rows_v, sem_g).wait()
    for g in range(_EPT // 16):
        rows = lax.iota(jnp.int32, 16) + g * 16
        cols = off_v[pl.ds(g * 16, 16)]
        vals = val_v[pl.ds(g * 16, 16)]
        plsc.addupdate_scatter(rows_v, [rows, cols], vals)
    pltpu.sync_copy(rows_v, patch.at[pl.ds(wid * _EPT, _EPT)])


def _tc_body(y_s, lo_s, hi_s, P_ref, Yb_ref, out_ref):
    b = pl.program_id(0)
    rb = pl.program_id(1)
    out_ref[...] = Yb_ref[...]

    def upd(k, carry):
        local = y_s[k] - rb * _RB
        out_ref[0, pl.ds(local, 1), :] = P_ref[0, pl.ds(k, 1), :]
        return carry

    jax.lax.fori_loop(lo_s[rb], hi_s[rb], upd, 0)


def kernel(Y, X, y_idx, x_idx):
    bb = jnp.repeat(jnp.arange(_B, dtype=jnp.int32), _K)
    yk = jnp.tile(y_idx, (_B,))
    xk = jnp.tile(x_idx, (_B,))
    row_e = (bb * _H + yk).reshape(_NW, _EPT)
    off_e = xk.reshape(_NW, _EPT)
    val_e = X.reshape(_NW, _EPT)

    patch = _sc_patch(Y.reshape(_B * _H, _W), row_e, off_e, val_e)

    edges = jnp.arange(_NRB, dtype=jnp.int32) * _RB
    lo = jnp.searchsorted(y_idx, edges).astype(jnp.int32)
    hi = jnp.searchsorted(y_idx, edges + _RB).astype(jnp.int32)

    out = pl.pallas_call(
        _tc_body,
        grid_spec=pltpu.PrefetchScalarGridSpec(
            num_scalar_prefetch=3,
            grid=(_B, _NRB),
            in_specs=[
                pl.BlockSpec((1, _K, _W), lambda b, rb, *_: (b, 0, 0)),
                pl.BlockSpec((1, _RB, _W), lambda b, rb, *_: (b, rb, 0)),
            ],
            out_specs=pl.BlockSpec((1, _RB, _W), lambda b, rb, *_: (b, rb, 0)),
        ),
        out_shape=jax.ShapeDtypeStruct((_B, _H, _W), jnp.float32),
    )(y_idx, lo, hi, patch.reshape(_B, _K, _W), Y)
    return out
